# baseline (device time: 76593 ns/iter reference)
import jax
import jax.numpy as jnp
from jax import lax
from jax.experimental import pallas as pl
from jax.experimental.pallas import tpu as pltpu

N_GLOBAL = 4096
EPS = 1e-5
BM = 512


def kernel(x, gamma, beta):
    m, n_loc = x.shape
    nb = m // BM

    def stats_body(x_ref, stats_ref, acc_ref, recv_ref, send_sem, recv_sem):
        i = pl.program_id(0)
        xb = x_ref[...]
        acc_ref[0, pl.ds(i * BM, BM)] = jnp.sum(xb, axis=1)
        acc_ref[1, pl.ds(i * BM, BM)] = jnp.sum(xb * xb, axis=1)

        @pl.when(i == nb - 1)
        def _():
            my_x = lax.axis_index("x")
            my_y = lax.axis_index("y")
            nbr = (my_x, 1 - my_y)
            barrier = pltpu.get_barrier_semaphore()
            pl.semaphore_signal(
                barrier, inc=1, device_id=nbr,
                device_id_type=pl.DeviceIdType.MESH,
            )
            pl.semaphore_wait(barrier, 1)
            rdma = pltpu.make_async_remote_copy(
                src_ref=acc_ref,
                dst_ref=recv_ref,
                send_sem=send_sem,
                recv_sem=recv_sem,
                device_id=nbr,
                device_id_type=pl.DeviceIdType.MESH,
            )
            rdma.start()
            rdma.wait()
            tot_s = acc_ref[0, :] + recv_ref[0, :]
            tot_ss = acc_ref[1, :] + recv_ref[1, :]
            mean = tot_s * (1.0 / N_GLOBAL)
            var = tot_ss * (1.0 / N_GLOBAL) - mean * mean
            stats_ref[0, :] = mean
            stats_ref[1, :] = lax.rsqrt(var + EPS)

    stats = pl.pallas_call(
        stats_body,
        grid=(nb,),
        in_specs=[pl.BlockSpec((BM, n_loc), lambda i: (i, 0))],
        out_specs=pl.BlockSpec((2, m), lambda i: (0, 0)),
        out_shape=jax.ShapeDtypeStruct((2, m), jnp.float32),
        scratch_shapes=[
            pltpu.VMEM((2, m), jnp.float32),
            pltpu.VMEM((2, m), jnp.float32),
            pltpu.SemaphoreType.DMA,
            pltpu.SemaphoreType.DMA,
        ],
        compiler_params=pltpu.CompilerParams(collective_id=0),
    )(x)

    def norm_body(x_ref, g_ref, b_ref, stats_ref, o_ref):
        i = pl.program_id(0)
        mean = stats_ref[0, pl.ds(i * BM, BM)][:, None]
        rstd = stats_ref[1, pl.ds(i * BM, BM)][:, None]
        o_ref[...] = g_ref[...] * ((x_ref[...] - mean) * rstd) + b_ref[...]

    out = pl.pallas_call(
        norm_body,
        grid=(nb,),
        in_specs=[
            pl.BlockSpec((BM, n_loc), lambda i: (i, 0)),
            pl.BlockSpec((1, n_loc), lambda i: (0, 0)),
            pl.BlockSpec((1, n_loc), lambda i: (0, 0)),
            pl.BlockSpec((2, m), lambda i: (0, 0)),
        ],
        out_specs=pl.BlockSpec((BM, n_loc), lambda i: (i, 0)),
        out_shape=jax.ShapeDtypeStruct((m, n_loc), jnp.float32),
    )(x, gamma.reshape(1, n_loc), beta.reshape(1, n_loc), stats)
    return out
